# baseline (speedup 1.0000x reference)
import functools

import jax
import jax.numpy as jnp
from jax import lax
from jax.experimental import pallas as pl
from jax.experimental.pallas import tpu as pltpu
from jax.experimental.pallas import tpu_sc as plsc

N = 10000
DIM = 128
E = 320000

NC = 2
NS = 16

EW = E // (NC * NS)
C1 = 80
NCH1 = EW // C1
NCH1E = NCH1 + 1

OWN = 320
CAP = 11136
C3 = 48
NCH3 = CAP // C3
SCH = 2000
NSCH = E // SCH
ACC_ROWS1 = 321
F2 = 16


def _proj_body(x_ref, c_ref, wq_ref, bq_ref, wk_ref, bk_ref, wv_ref, bv_ref,
               r_ref, ct_ref, t_ref):
    xb = x_ref[...]
    cb = c_ref[...]
    q = jnp.dot(xb, wq_ref[...], preferred_element_type=jnp.float32) + bq_ref[...]
    k = jnp.dot(xb, wk_ref[...], preferred_element_type=jnp.float32) + bk_ref[...]
    v = jnp.dot(xb, wv_ref[...], preferred_element_type=jnp.float32) + bv_ref[...]
    r_ref[:, :DIM] = q
    r_ref[:, DIM:] = cb
    ct_ref[:, :DIM] = k
    ct_ref[:, DIM:] = cb
    t_ref[:, :DIM] = v
    t_ref[:, DIM:] = cb


def _proj(x, coord, Wq, bq, Wk, bk, Wv, bv):
    BLK = 1000
    grid = (N // BLK,)
    row_spec = pl.BlockSpec((BLK, DIM), lambda i: (i, 0))
    out_spec = pl.BlockSpec((BLK, 2 * DIM), lambda i: (i, 0))
    w_spec = pl.BlockSpec((DIM, DIM), lambda i: (0, 0))
    b_spec = pl.BlockSpec((1, DIM), lambda i: (0, 0))
    return pl.pallas_call(
        _proj_body,
        grid=grid,
        in_specs=[row_spec, row_spec, w_spec, b_spec, w_spec, b_spec, w_spec,
                  b_spec],
        out_specs=[out_spec, out_spec, out_spec],
        out_shape=[jax.ShapeDtypeStruct((N, 2 * DIM), jnp.float32)] * 3,
    )(x, coord, Wq, bq.reshape(1, DIM), Wk, bk.reshape(1, DIM), Wv,
      bv.reshape(1, DIM))


def _soft_body(l_ref, s_ref, w_ref, t0_ref):
    l = l_ref[...]
    m = jnp.max(l)
    e = jnp.exp(l - m)
    z = jnp.sum(e)
    w = e / z
    w_ref[...] = w
    t0_ref[...] = w * jnp.sqrt(s_ref[...])


def _soft(logits, sumsq):
    R = E // 128
    l2 = logits.reshape(R, 128)
    s2 = sumsq.reshape(R, 128)
    w, t0 = pl.pallas_call(
        _soft_body,
        out_shape=[jax.ShapeDtypeStruct((R, 128), jnp.float32)] * 2,
    )(l2, s2)
    return w.reshape(E), t0.reshape(E)


def _p1_body(row_hbm, col_hbm, r_hbm, ct_hbm, logit_hbm, sumsq_hbm,
             rowv0, rowv1, colv0, colv1, rbuf0, rbuf1, cbuf0, cbuf1,
             lbuf, sbuf, m1, m2, semA0, semA1, semB0, semB1):
    c = lax.axis_index("c")
    s = lax.axis_index("s")
    wid = s * NC + c
    base = wid * EW
    lane = lax.iota(jnp.int32, 16)
    zero16 = jnp.zeros((16,), jnp.float32)

    rowv = [rowv0, rowv1]
    colv = [colv0, colv1]
    rbuf = [rbuf0, rbuf1]
    cbuf = [cbuf0, cbuf1]
    semA = [semA0, semA1]
    semB = [semB0, semB1]

    def eoff(i):
        return base + lax.rem(i, NCH1) * C1

    def fire_s1(i, b):
        eb = eoff(i)
        pltpu.async_copy(row_hbm.at[pl.ds(eb, C1)], rowv[b], semA[b])
        pltpu.async_copy(col_hbm.at[pl.ds(eb, C1)], colv[b], semA[b])

    def drain_s1(i, b):
        eb = eoff(i)
        pltpu.make_async_copy(row_hbm.at[pl.ds(eb, C1)], rowv[b], semA[b]).wait()
        pltpu.make_async_copy(col_hbm.at[pl.ds(eb, C1)], colv[b], semA[b]).wait()

    def fire_s2(i, b):
        pltpu.async_copy(r_hbm.at[rowv[b]], rbuf[b], semB[b])
        pltpu.async_copy(ct_hbm.at[colv[b]], cbuf[b], semB[b])

    def drain_s2(i, b):
        pltpu.make_async_copy(r_hbm.at[rowv[b]], rbuf[b], semB[b]).wait()
        pltpu.make_async_copy(ct_hbm.at[colv[b]], cbuf[b], semB[b]).wait()

    def compute(i, b):
        out0 = lax.rem(i, NCH1) * C1
        rb = rbuf[b]
        cb = cbuf[b]

        def group(g, _):
            def edge(k, _):
                e = g * 16 + k
                acc1 = zero16
                acc2 = zero16
                for j in range(8):
                    qv = rb[e, pl.ds(16 * j, 16)]
                    kv = cb[e, pl.ds(16 * j, 16)]
                    acc1 = acc1 + qv * kv
                    cr = rb[e, pl.ds(DIM + 16 * j, 16)]
                    cc = cb[e, pl.ds(DIM + 16 * j, 16)]
                    d = cr - cc
                    acc2 = acc2 + d * d
                m1[k, pl.ds(0, 16)] = acc1
                m2[k, pl.ds(0, 16)] = acc2
                return 0

            lax.fori_loop(0, 16, edge, 0, unroll=2)
            suml = zero16
            sums = zero16
            for j in range(16):
                cj = jnp.full((16,), j, jnp.int32)
                suml = suml + plsc.load_gather(m1, [lane, cj])
                sums = sums + plsc.load_gather(m2, [lane, cj])
            lbuf[pl.ds(out0 + g * 16, 16)] = suml
            sbuf[pl.ds(out0 + g * 16, 16)] = sums
            return 0

        lax.fori_loop(0, C1 // 16, group, 0)

    fire_s1(0, 0)
    drain_s1(0, 0)
    fire_s2(0, 0)
    fire_s1(1, 1)

    def pair(g, _):
        for b in (0, 1):
            i = 2 * g + b
            drain_s1(i + 1, 1 - b)
            fire_s2(i + 1, 1 - b)
            drain_s2(i, b)
            compute(i, b)
            fire_s1(i + 2, b)
        return 0

    lax.fori_loop(0, NCH1E // 2, pair, 0)

    drain_s2(NCH1E, 0)
    drain_s1(NCH1E + 1, 1)

    pltpu.sync_copy(lbuf, logit_hbm.at[pl.ds(base, EW)])
    pltpu.sync_copy(sbuf, sumsq_hbm.at[pl.ds(base, EW)])


_p1 = functools.partial(
    pl.kernel,
    out_type=[jax.ShapeDtypeStruct((E,), jnp.float32)] * 2,
    mesh=plsc.VectorSubcoreMesh(core_axis_name="c", subcore_axis_name="s"),
    compiler_params=pltpu.CompilerParams(needs_layout_passes=False),
    scratch_types=[
        pltpu.VMEM((C1,), jnp.int32),
        pltpu.VMEM((C1,), jnp.int32),
        pltpu.VMEM((C1,), jnp.int32),
        pltpu.VMEM((C1,), jnp.int32),
        pltpu.VMEM((C1, 2 * DIM), jnp.float32),
        pltpu.VMEM((C1, 2 * DIM), jnp.float32),
        pltpu.VMEM((C1, 2 * DIM), jnp.float32),
        pltpu.VMEM((C1, 2 * DIM), jnp.float32),
        pltpu.VMEM((EW,), jnp.float32),
        pltpu.VMEM((EW,), jnp.float32),
        pltpu.VMEM((16, 16), jnp.float32),
        pltpu.VMEM((16, 16), jnp.float32),
        pltpu.SemaphoreType.DMA,
        pltpu.SemaphoreType.DMA,
        pltpu.SemaphoreType.DMA,
        pltpu.SemaphoreType.DMA,
    ],
)(_p1_body)


def _p2_body(row_hbm, col_hbm, w_hbm, t0_hbm, t_hbm, x_hbm, coord_hbm,
             wc_hbm, bc_hbm, xnew_hbm, cnew_hbm,
             eidlist, rowbuf,
             eidv0, eidv1, eidv2, eidv3, colv0, colv1, colv2, colv3,
             wv0, wv1, wv2, wv3, t0v0, t0v1, t0v2, t0v3, lrv,
             tbuf0, tbuf1, fxb, fcb, wcb, bcb, acc1, abacc,
             semA0, semA1, semA2, semA3, semW0, semW1, semW2, semW3,
             semT0, semT1, semT2, semT3, semB0, semB1):
    c = lax.axis_index("c")
    s = lax.axis_index("s")
    wid = c * NS + s
    lane = lax.iota(jnp.int32, 16)
    zero16 = jnp.zeros((16,), jnp.float32)
    izero16 = jnp.zeros((16,), jnp.int32)

    eidv = [eidv0, eidv1, eidv2, eidv3]
    colv = [colv0, colv1, colv2, colv3]
    wv = [wv0, wv1, wv2, wv3]
    t0v = [t0v0, t0v1, t0v2, t0v3]
    tbuf = [tbuf0, tbuf1]
    semA = [semA0, semA1, semA2, semA3]
    semW = [semW0, semW1, semW2, semW3]
    semT = [semT0, semT1, semT2, semT3]
    semB = [semB0, semB1]

    def za(i, _):
        acc1[pl.ds(i * 16, 16)] = zero16
        return 0

    lax.fori_loop(0, (ACC_ROWS1 * 2 * DIM) // 16, za, 0)

    def zb(i, _):
        abacc[pl.ds(i * 16, 16)] = zero16
        return 0

    lax.fori_loop(0, (2 * ACC_ROWS1 + 14) // 16, zb, 0)

    def ze(i, _):
        eidlist[pl.ds(i * 16, 16)] = izero16
        return 0

    lax.fori_loop(0, CAP // 16, ze, 0)

    pltpu.sync_copy(wc_hbm, wcb)
    pltpu.sync_copy(bc_hbm, bcb)
    wc_regs = [wcb[pl.ds(16 * j, 16)] for j in range(8)]
    bc_regs = [bcb[pl.ds(16 * j, 16)] for j in range(8)]

    def scan_chunk(ich, cnt):
        pltpu.sync_copy(row_hbm.at[pl.ds(ich * SCH, SCH)], rowbuf)

        def g(gi, cnt):
            r = rowbuf[pl.ds(gi * 16, 16)]
            msk = (r // OWN) == wid
            pre = plsc.cumsum(jnp.where(msk, 1, 0))
            pos = cnt + pre - 1
            msk = msk & (pos < CAP)
            eidvec = ich * SCH + gi * 16 + lane
            pk = eidvec * 512 + (r - wid * OWN)
            plsc.store_scatter(eidlist, [pos], pk, mask=msk)
            return cnt + plsc.all_reduce_population_count(msk)

        return lax.fori_loop(0, SCH // 16, g, cnt)

    cnt = lax.fori_loop(0, NSCH, scan_chunk, izero16)

    def fire_s1(i, b):
        b0 = lax.rem(i, NCH3) * C3
        for o in (0, 16, 32):
            pk = eidlist[pl.ds(b0 + o, 16)]
            eidv[b][pl.ds(o, 16)] = pk // 512
        es = eidv[b]
        pltpu.async_copy(col_hbm.at[es], colv[b], semA[b])
        pltpu.async_copy(w_hbm.at[es], wv[b], semW[b])
        pltpu.async_copy(t0_hbm.at[es], t0v[b], semT[b])

    def drain_s1(i, b):
        es = eidv[b]
        pltpu.make_async_copy(col_hbm.at[es], colv[b], semA[b]).wait()
        pltpu.make_async_copy(w_hbm.at[es], wv[b], semW[b]).wait()
        pltpu.make_async_copy(t0_hbm.at[es], t0v[b], semT[b]).wait()

    def fire_s2(i, b4, t):
        pltpu.async_copy(t_hbm.at[colv[b4]], tbuf[t], semB[t])

    def drain_s2(i, b4, t):
        pltpu.make_async_copy(t_hbm.at[colv[b4]], tbuf[t], semB[t]).wait()

    def compute(i, b4, t):
        base = lax.rem(i, NCH3) * C3
        tb = tbuf[t]
        wb = wv[b4]
        t0b = t0v[b4]

        for o in (0, 16, 32):
            posv = base + o + lane
            pk = eidlist[pl.ds(base + o, 16)]
            lr = lax.rem(pk, 512)
            lrv[pl.ds(o, 16)] = jnp.where(posv < cnt, lr, OWN)

        def edge(e, _):
            esplat = jnp.broadcast_to(e, (16,)).astype(jnp.int32)
            w_s = plsc.load_gather(wb, [esplat])
            t0_s = plsc.load_gather(t0b, [esplat])
            lr_s = plsc.load_gather(lrv, [esplat])
            rb = lr_s * (2 * DIM) + lane
            for j in range(8):
                vvj = tb[e, pl.ds(16 * j, 16)]
                plsc.addupdate_scatter(acc1, [rb + 16 * j], vvj * w_s)
                cc = tb[e, pl.ds(DIM + 16 * j, 16)]
                coef = t0_s * wc_regs[j] + w_s * bc_regs[j]
                plsc.addupdate_scatter(acc1, [rb + (DIM + 16 * j)],
                                       cc * coef)
            abvec = jnp.where(lane == 0, t0_s,
                              jnp.where(lane == 1, w_s, 0.0))
            plsc.addupdate_scatter(abacc, [lr_s * 2 + lane], abvec,
                                   mask=lane < 2)
            return 0

        lax.fori_loop(0, C3, edge, 0, unroll=4)

    fire_s1(0, 0)
    drain_s1(0, 0)
    fire_s2(0, 0, 0)
    fire_s1(1, 1)
    fire_s1(2, 2)
    fire_s1(3, 3)

    def quad(g, _):
        for u in range(4):
            i = 4 * g + u
            b1 = (u + 1) % 4
            t1 = (u + 1) % 2
            drain_s1(i + 1, b1)
            fire_s2(i + 1, b1, t1)
            drain_s2(i, u % 4, u % 2)
            compute(i, u % 4, u % 2)
            fire_s1(i + 4, u % 4)
        return 0

    lax.fori_loop(0, NCH3 // 4, quad, 0)

    drain_s2(NCH3, 0, 0)
    drain_s1(NCH3 + 1, 1)
    drain_s1(NCH3 + 2, 2)
    drain_s1(NCH3 + 3, 3)

    def fchunk(k, _):
        g0 = wid * OWN + k * F2

        @pl.when(g0 < N)
        def _():
            pltpu.sync_copy(x_hbm.at[pl.ds(g0, F2)], fxb)
            pltpu.sync_copy(coord_hbm.at[pl.ds(g0, F2)], fcb)

            def node(r, _):
                lr = k * F2 + r
                ab = lr * (2 * DIM)
                absp = jnp.broadcast_to(lr * 2, (16,)).astype(jnp.int32)
                A = plsc.load_gather(abacc, [absp])
                B = plsc.load_gather(abacc, [absp + 1])
                for j in range(8):
                    sl = pl.ds(16 * j, 16)
                    fxb[r, sl] = fxb[r, sl] + acc1[pl.ds(ab + 16 * j, 16)]
                    cj = fcb[r, sl]
                    fcb[r, sl] = (cj * (1.0 + wc_regs[j] * A + bc_regs[j] * B)
                                  - acc1[pl.ds(ab + DIM + 16 * j, 16)])
                return 0

            lax.fori_loop(0, F2, node, 0)
            pltpu.sync_copy(fxb, xnew_hbm.at[pl.ds(g0, F2)])
            pltpu.sync_copy(fcb, cnew_hbm.at[pl.ds(g0, F2)])

        return 0

    lax.fori_loop(0, OWN // F2, fchunk, 0)


_p2 = functools.partial(
    pl.kernel,
    out_type=[jax.ShapeDtypeStruct((N, DIM), jnp.float32)] * 2,
    mesh=plsc.VectorSubcoreMesh(core_axis_name="c", subcore_axis_name="s"),
    compiler_params=pltpu.CompilerParams(needs_layout_passes=False),
    scratch_types=(
        [pltpu.VMEM((CAP,), jnp.int32),
         pltpu.VMEM((SCH,), jnp.int32)]
        + [pltpu.VMEM((C3,), jnp.int32) for _ in range(4)]
        + [pltpu.VMEM((C3,), jnp.int32) for _ in range(4)]
        + [pltpu.VMEM((C3,), jnp.float32) for _ in range(4)]
        + [pltpu.VMEM((C3,), jnp.float32) for _ in range(4)]
        + [pltpu.VMEM((C3,), jnp.int32),
           pltpu.VMEM((C3, 2 * DIM), jnp.float32),
           pltpu.VMEM((C3, 2 * DIM), jnp.float32),
           pltpu.VMEM((F2, DIM), jnp.float32),
           pltpu.VMEM((F2, DIM), jnp.float32),
           pltpu.VMEM((DIM,), jnp.float32),
           pltpu.VMEM((DIM,), jnp.float32),
           pltpu.VMEM((ACC_ROWS1 * 2 * DIM,), jnp.float32),
           pltpu.VMEM((2 * ACC_ROWS1 + 14,), jnp.float32)]
        + [pltpu.SemaphoreType.DMA for _ in range(14)]
    ),
)(_p2_body)


def kernel(x, coord, edge_index, Wq, bq, Wk, bk, Wv, bv, Wc, bc):
    row = edge_index[0]
    col = edge_index[1]
    R, CT, T = _proj(x, coord, Wq, bq, Wk, bk, Wv, bv)
    logits, sumsq = _p1(row, col, R, CT)
    w, t0 = _soft(logits, sumsq)
    x_new, coord_new = _p2(row, col, w, t0, T, x, coord,
                           Wc.reshape(DIM), bc)
    return (x_new, coord_new)


# speedup vs baseline: 1.0039x; 1.0039x over previous
import functools

import jax
import jax.numpy as jnp
from jax import lax
from jax.experimental import pallas as pl
from jax.experimental.pallas import tpu as pltpu
from jax.experimental.pallas import tpu_sc as plsc

N = 10000
DIM = 128
E = 320000

NC = 2
NS = 16

EW = E // (NC * NS)
C1 = 80
NCH1 = EW // C1
NCH1E = NCH1 + 1

OWN = 320
CAP = 11136
C3 = 48
NCH3 = CAP // C3
SCH = 2000
NSCH = E // SCH
ACC_ROWS1 = 321
F2 = 16


def _proj_body(x_ref, c_ref, wq_ref, bq_ref, wk_ref, bk_ref, wv_ref, bv_ref,
               r_ref, ct_ref, t_ref):
    xb = x_ref[...]
    cb = c_ref[...]
    q = jnp.dot(xb, wq_ref[...], preferred_element_type=jnp.float32) + bq_ref[...]
    k = jnp.dot(xb, wk_ref[...], preferred_element_type=jnp.float32) + bk_ref[...]
    v = jnp.dot(xb, wv_ref[...], preferred_element_type=jnp.float32) + bv_ref[...]
    r_ref[:, :DIM] = q
    r_ref[:, DIM:] = cb
    ct_ref[:, :DIM] = k
    ct_ref[:, DIM:] = cb
    t_ref[:, :DIM] = v
    t_ref[:, DIM:] = cb


def _proj(x, coord, Wq, bq, Wk, bk, Wv, bv):
    BLK = 1000
    grid = (N // BLK,)
    row_spec = pl.BlockSpec((BLK, DIM), lambda i: (i, 0))
    out_spec = pl.BlockSpec((BLK, 2 * DIM), lambda i: (i, 0))
    w_spec = pl.BlockSpec((DIM, DIM), lambda i: (0, 0))
    b_spec = pl.BlockSpec((1, DIM), lambda i: (0, 0))
    return pl.pallas_call(
        _proj_body,
        grid=grid,
        in_specs=[row_spec, row_spec, w_spec, b_spec, w_spec, b_spec, w_spec,
                  b_spec],
        out_specs=[out_spec, out_spec, out_spec],
        out_shape=[jax.ShapeDtypeStruct((N, 2 * DIM), jnp.float32)] * 3,
    )(x, coord, Wq, bq.reshape(1, DIM), Wk, bk.reshape(1, DIM), Wv,
      bv.reshape(1, DIM))


def _soft_body(l_ref, s_ref, w_ref, t0_ref):
    l = l_ref[...]
    m = jnp.max(l)
    e = jnp.exp(l - m)
    z = jnp.sum(e)
    w = e / z
    w_ref[...] = w
    t0_ref[...] = w * jnp.sqrt(s_ref[...])


def _soft(logits, sumsq):
    R = E // 128
    l2 = logits.reshape(R, 128)
    s2 = sumsq.reshape(R, 128)
    w, t0 = pl.pallas_call(
        _soft_body,
        out_shape=[jax.ShapeDtypeStruct((R, 128), jnp.float32)] * 2,
    )(l2, s2)
    return w.reshape(E), t0.reshape(E)


def _p1_body(row_hbm, col_hbm, r_hbm, ct_hbm, logit_hbm, sumsq_hbm,
             rowv0, rowv1, colv0, colv1, rbuf0, rbuf1, cbuf0, cbuf1,
             lbuf, sbuf, m1, m2, semA0, semA1, semB0, semB1):
    c = lax.axis_index("c")
    s = lax.axis_index("s")
    wid = s * NC + c
    base = wid * EW
    lane = lax.iota(jnp.int32, 16)
    zero16 = jnp.zeros((16,), jnp.float32)

    rowv = [rowv0, rowv1]
    colv = [colv0, colv1]
    rbuf = [rbuf0, rbuf1]
    cbuf = [cbuf0, cbuf1]
    semA = [semA0, semA1]
    semB = [semB0, semB1]

    def eoff(i):
        return base + lax.rem(i, NCH1) * C1

    def fire_s1(i, b):
        eb = eoff(i)
        pltpu.async_copy(row_hbm.at[pl.ds(eb, C1)], rowv[b], semA[b])
        pltpu.async_copy(col_hbm.at[pl.ds(eb, C1)], colv[b], semA[b])

    def drain_s1(i, b):
        eb = eoff(i)
        pltpu.make_async_copy(row_hbm.at[pl.ds(eb, C1)], rowv[b], semA[b]).wait()
        pltpu.make_async_copy(col_hbm.at[pl.ds(eb, C1)], colv[b], semA[b]).wait()

    def fire_s2(i, b):
        pltpu.async_copy(r_hbm.at[rowv[b]], rbuf[b], semB[b])
        pltpu.async_copy(ct_hbm.at[colv[b]], cbuf[b], semB[b])

    def drain_s2(i, b):
        pltpu.make_async_copy(r_hbm.at[rowv[b]], rbuf[b], semB[b]).wait()
        pltpu.make_async_copy(ct_hbm.at[colv[b]], cbuf[b], semB[b]).wait()

    def compute(i, b):
        out0 = lax.rem(i, NCH1) * C1
        rb = rbuf[b]
        cb = cbuf[b]

        def group(g, _):
            def edge(k, _):
                e = g * 16 + k
                acc1 = zero16
                acc2 = zero16
                for j in range(8):
                    qv = rb[e, pl.ds(16 * j, 16)]
                    kv = cb[e, pl.ds(16 * j, 16)]
                    acc1 = acc1 + qv * kv
                    cr = rb[e, pl.ds(DIM + 16 * j, 16)]
                    cc = cb[e, pl.ds(DIM + 16 * j, 16)]
                    d = cr - cc
                    acc2 = acc2 + d * d
                m1[k, pl.ds(0, 16)] = acc1
                m2[k, pl.ds(0, 16)] = acc2
                return 0

            lax.fori_loop(0, 16, edge, 0)
            suml = zero16
            sums = zero16
            for j in range(16):
                cj = jnp.full((16,), j, jnp.int32)
                suml = suml + plsc.load_gather(m1, [lane, cj])
                sums = sums + plsc.load_gather(m2, [lane, cj])
            lbuf[pl.ds(out0 + g * 16, 16)] = suml
            sbuf[pl.ds(out0 + g * 16, 16)] = sums
            return 0

        lax.fori_loop(0, C1 // 16, group, 0)

    fire_s1(0, 0)
    drain_s1(0, 0)
    fire_s2(0, 0)
    fire_s1(1, 1)

    def pair(g, _):
        for b in (0, 1):
            i = 2 * g + b
            drain_s1(i + 1, 1 - b)
            fire_s2(i + 1, 1 - b)
            drain_s2(i, b)
            compute(i, b)
            fire_s1(i + 2, b)
        return 0

    lax.fori_loop(0, NCH1E // 2, pair, 0)

    drain_s2(NCH1E, 0)
    drain_s1(NCH1E + 1, 1)

    pltpu.sync_copy(lbuf, logit_hbm.at[pl.ds(base, EW)])
    pltpu.sync_copy(sbuf, sumsq_hbm.at[pl.ds(base, EW)])


_p1 = functools.partial(
    pl.kernel,
    out_type=[jax.ShapeDtypeStruct((E,), jnp.float32)] * 2,
    mesh=plsc.VectorSubcoreMesh(core_axis_name="c", subcore_axis_name="s"),
    compiler_params=pltpu.CompilerParams(needs_layout_passes=False),
    scratch_types=[
        pltpu.VMEM((C1,), jnp.int32),
        pltpu.VMEM((C1,), jnp.int32),
        pltpu.VMEM((C1,), jnp.int32),
        pltpu.VMEM((C1,), jnp.int32),
        pltpu.VMEM((C1, 2 * DIM), jnp.float32),
        pltpu.VMEM((C1, 2 * DIM), jnp.float32),
        pltpu.VMEM((C1, 2 * DIM), jnp.float32),
        pltpu.VMEM((C1, 2 * DIM), jnp.float32),
        pltpu.VMEM((EW,), jnp.float32),
        pltpu.VMEM((EW,), jnp.float32),
        pltpu.VMEM((16, 16), jnp.float32),
        pltpu.VMEM((16, 16), jnp.float32),
        pltpu.SemaphoreType.DMA,
        pltpu.SemaphoreType.DMA,
        pltpu.SemaphoreType.DMA,
        pltpu.SemaphoreType.DMA,
    ],
)(_p1_body)


def _p2_body(row_hbm, col_hbm, w_hbm, t0_hbm, t_hbm, x_hbm, coord_hbm,
             wc_hbm, bc_hbm, xnew_hbm, cnew_hbm,
             eidlist, rowbuf,
             eidv0, eidv1, eidv2, eidv3, colv0, colv1, colv2, colv3,
             wv0, wv1, wv2, wv3, t0v0, t0v1, t0v2, t0v3, lrv,
             tbuf0, tbuf1, fxb, fcb, wcb, bcb, acc1, abacc,
             semA0, semA1, semA2, semA3, semW0, semW1, semW2, semW3,
             semT0, semT1, semT2, semT3, semB0, semB1):
    c = lax.axis_index("c")
    s = lax.axis_index("s")
    wid = c * NS + s
    lane = lax.iota(jnp.int32, 16)
    zero16 = jnp.zeros((16,), jnp.float32)
    izero16 = jnp.zeros((16,), jnp.int32)

    eidv = [eidv0, eidv1, eidv2, eidv3]
    colv = [colv0, colv1, colv2, colv3]
    wv = [wv0, wv1, wv2, wv3]
    t0v = [t0v0, t0v1, t0v2, t0v3]
    tbuf = [tbuf0, tbuf1]
    semA = [semA0, semA1, semA2, semA3]
    semW = [semW0, semW1, semW2, semW3]
    semT = [semT0, semT1, semT2, semT3]
    semB = [semB0, semB1]

    def za(i, _):
        acc1[pl.ds(i * 16, 16)] = zero16
        return 0

    lax.fori_loop(0, (ACC_ROWS1 * 2 * DIM) // 16, za, 0)

    def zb(i, _):
        abacc[pl.ds(i * 16, 16)] = zero16
        return 0

    lax.fori_loop(0, (2 * ACC_ROWS1 + 14) // 16, zb, 0)

    def ze(i, _):
        eidlist[pl.ds(i * 16, 16)] = izero16
        return 0

    lax.fori_loop(0, CAP // 16, ze, 0)

    pltpu.sync_copy(wc_hbm, wcb)
    pltpu.sync_copy(bc_hbm, bcb)
    wc_regs = [wcb[pl.ds(16 * j, 16)] for j in range(8)]
    bc_regs = [bcb[pl.ds(16 * j, 16)] for j in range(8)]

    def scan_chunk(ich, cnt):
        pltpu.sync_copy(row_hbm.at[pl.ds(ich * SCH, SCH)], rowbuf)

        def g(gi, cnt):
            r = rowbuf[pl.ds(gi * 16, 16)]
            msk = (r // OWN) == wid
            pre = plsc.cumsum(jnp.where(msk, 1, 0))
            pos = cnt + pre - 1
            msk = msk & (pos < CAP)
            eidvec = ich * SCH + gi * 16 + lane
            pk = eidvec * 512 + (r - wid * OWN)
            plsc.store_scatter(eidlist, [pos], pk, mask=msk)
            return cnt + plsc.all_reduce_population_count(msk)

        return lax.fori_loop(0, SCH // 16, g, cnt)

    cnt = lax.fori_loop(0, NSCH, scan_chunk, izero16)

    def fire_s1(i, b):
        b0 = lax.rem(i, NCH3) * C3
        for o in (0, 16, 32):
            pk = eidlist[pl.ds(b0 + o, 16)]
            eidv[b][pl.ds(o, 16)] = pk // 512
        es = eidv[b]
        pltpu.async_copy(col_hbm.at[es], colv[b], semA[b])
        pltpu.async_copy(w_hbm.at[es], wv[b], semW[b])
        pltpu.async_copy(t0_hbm.at[es], t0v[b], semT[b])

    def drain_s1(i, b):
        es = eidv[b]
        pltpu.make_async_copy(col_hbm.at[es], colv[b], semA[b]).wait()
        pltpu.make_async_copy(w_hbm.at[es], wv[b], semW[b]).wait()
        pltpu.make_async_copy(t0_hbm.at[es], t0v[b], semT[b]).wait()

    def fire_s2(i, b4, t):
        pltpu.async_copy(t_hbm.at[colv[b4]], tbuf[t], semB[t])

    def drain_s2(i, b4, t):
        pltpu.make_async_copy(t_hbm.at[colv[b4]], tbuf[t], semB[t]).wait()

    def compute(i, b4, t):
        base = lax.rem(i, NCH3) * C3
        tb = tbuf[t]
        wb = wv[b4]
        t0b = t0v[b4]

        for o in (0, 16, 32):
            posv = base + o + lane
            pk = eidlist[pl.ds(base + o, 16)]
            lr = lax.rem(pk, 512)
            lrv[pl.ds(o, 16)] = jnp.where(posv < cnt, lr, OWN)

        def edge(e, _):
            esplat = jnp.broadcast_to(e, (16,)).astype(jnp.int32)
            w_s = plsc.load_gather(wb, [esplat])
            t0_s = plsc.load_gather(t0b, [esplat])
            lr_s = plsc.load_gather(lrv, [esplat])
            rb = lr_s * (2 * DIM) + lane
            for j in range(8):
                vvj = tb[e, pl.ds(16 * j, 16)]
                plsc.addupdate_scatter(acc1, [rb + 16 * j], vvj * w_s)
                cc = tb[e, pl.ds(DIM + 16 * j, 16)]
                coef = t0_s * wc_regs[j] + w_s * bc_regs[j]
                plsc.addupdate_scatter(acc1, [rb + (DIM + 16 * j)],
                                       cc * coef)
            abvec = jnp.where(lane == 0, t0_s,
                              jnp.where(lane == 1, w_s, 0.0))
            plsc.addupdate_scatter(abacc, [lr_s * 2 + lane], abvec,
                                   mask=lane < 2)
            return 0

        lax.fori_loop(0, C3, edge, 0)

    fire_s1(0, 0)
    drain_s1(0, 0)
    fire_s2(0, 0, 0)
    fire_s1(1, 1)
    fire_s1(2, 2)
    fire_s1(3, 3)

    def quad(g, _):
        for u in range(4):
            i = 4 * g + u
            b1 = (u + 1) % 4
            t1 = (u + 1) % 2
            drain_s1(i + 1, b1)
            fire_s2(i + 1, b1, t1)
            drain_s2(i, u % 4, u % 2)
            compute(i, u % 4, u % 2)
            fire_s1(i + 4, u % 4)
        return 0

    lax.fori_loop(0, NCH3 // 4, quad, 0)

    drain_s2(NCH3, 0, 0)
    drain_s1(NCH3 + 1, 1)
    drain_s1(NCH3 + 2, 2)
    drain_s1(NCH3 + 3, 3)

    def fchunk(k, _):
        g0 = wid * OWN + k * F2

        @pl.when(g0 < N)
        def _():
            pltpu.sync_copy(x_hbm.at[pl.ds(g0, F2)], fxb)
            pltpu.sync_copy(coord_hbm.at[pl.ds(g0, F2)], fcb)

            def node(r, _):
                lr = k * F2 + r
                ab = lr * (2 * DIM)
                absp = jnp.broadcast_to(lr * 2, (16,)).astype(jnp.int32)
                A = plsc.load_gather(abacc, [absp])
                B = plsc.load_gather(abacc, [absp + 1])
                for j in range(8):
                    sl = pl.ds(16 * j, 16)
                    fxb[r, sl] = fxb[r, sl] + acc1[pl.ds(ab + 16 * j, 16)]
                    cj = fcb[r, sl]
                    fcb[r, sl] = (cj * (1.0 + wc_regs[j] * A + bc_regs[j] * B)
                                  - acc1[pl.ds(ab + DIM + 16 * j, 16)])
                return 0

            lax.fori_loop(0, F2, node, 0)
            pltpu.sync_copy(fxb, xnew_hbm.at[pl.ds(g0, F2)])
            pltpu.sync_copy(fcb, cnew_hbm.at[pl.ds(g0, F2)])

        return 0

    lax.fori_loop(0, OWN // F2, fchunk, 0)


_p2 = functools.partial(
    pl.kernel,
    out_type=[jax.ShapeDtypeStruct((N, DIM), jnp.float32)] * 2,
    mesh=plsc.VectorSubcoreMesh(core_axis_name="c", subcore_axis_name="s"),
    compiler_params=pltpu.CompilerParams(needs_layout_passes=False),
    scratch_types=(
        [pltpu.VMEM((CAP,), jnp.int32),
         pltpu.VMEM((SCH,), jnp.int32)]
        + [pltpu.VMEM((C3,), jnp.int32) for _ in range(4)]
        + [pltpu.VMEM((C3,), jnp.int32) for _ in range(4)]
        + [pltpu.VMEM((C3,), jnp.float32) for _ in range(4)]
        + [pltpu.VMEM((C3,), jnp.float32) for _ in range(4)]
        + [pltpu.VMEM((C3,), jnp.int32),
           pltpu.VMEM((C3, 2 * DIM), jnp.float32),
           pltpu.VMEM((C3, 2 * DIM), jnp.float32),
           pltpu.VMEM((F2, DIM), jnp.float32),
           pltpu.VMEM((F2, DIM), jnp.float32),
           pltpu.VMEM((DIM,), jnp.float32),
           pltpu.VMEM((DIM,), jnp.float32),
           pltpu.VMEM((ACC_ROWS1 * 2 * DIM,), jnp.float32),
           pltpu.VMEM((2 * ACC_ROWS1 + 14,), jnp.float32)]
        + [pltpu.SemaphoreType.DMA for _ in range(14)]
    ),
)(_p2_body)


def kernel(x, coord, edge_index, Wq, bq, Wk, bk, Wv, bv, Wc, bc):
    row = edge_index[0]
    col = edge_index[1]
    R, CT, T = _proj(x, coord, Wq, bq, Wk, bk, Wv, bv)
    logits, sumsq = _p1(row, col, R, CT)
    w, t0 = _soft(logits, sumsq)
    x_new, coord_new = _p2(row, col, w, t0, T, x, coord,
                           Wc.reshape(DIM), bc)
    return (x_new, coord_new)
